# baseline (speedup 1.0000x reference)
import functools

import jax
import jax.numpy as jnp
from jax import lax
from jax.experimental import pallas as pl
from jax.experimental.pallas import tpu as pltpu
from jax.experimental.pallas import tpu_sc as plsc

_INFO = plsc.get_sparse_core_info()
_NC, _NS = _INFO.num_cores, _INFO.num_subcores
_NW = _NC * _NS
_CAP = 40
_NBUF = 3


def _chunks(rows):
    out, off = [], 0
    for sz in (8, 16, 24):
        if off + sz <= rows:
            out.append((off, sz))
            off += sz
    while off < rows:
        sz = min(_CAP, rows - off)
        out.append((off, sz))
        off += sz
    return out


def kernel(x, pos_emb):
    S = x.shape[0]
    D = pos_emb.shape[1]
    src = pos_emb[:S]
    rows_per_w = S // _NW
    chunks = _chunks(rows_per_w)
    nchunks = len(chunks)
    mesh = plsc.VectorSubcoreMesh(core_axis_name="c", subcore_axis_name="s")

    @functools.partial(
        pl.kernel,
        out_type=jax.ShapeDtypeStruct((S, 1, D), jnp.float32),
        mesh=mesh,
        scratch_types=[
            pltpu.VMEM((_NBUF, _CAP, D), jnp.float32),
            pltpu.SemaphoreType.DMA((_NBUF,)),
            pltpu.SemaphoreType.DMA((_NBUF,)),
        ],
    )
    def _copy(src_hbm, out_hbm, buf, rsem, wsem):
        wid = lax.axis_index("s") * _NC + lax.axis_index("c")
        base = wid * rows_per_w

        def read(i):
            off, sz = chunks[i]
            return pltpu.make_async_copy(
                src_hbm.at[pl.ds(base + off, sz)],
                buf.at[i % _NBUF, pl.ds(0, sz)],
                rsem.at[i % _NBUF],
            )

        def write(i):
            off, sz = chunks[i]
            return pltpu.make_async_copy(
                buf.at[i % _NBUF, pl.ds(0, sz)],
                out_hbm.at[pl.ds(base + off, sz), 0],
                wsem.at[i % _NBUF],
            )

        for i in range(min(_NBUF - 1, nchunks)):
            read(i).start()
        for i in range(nchunks):
            read(i).wait()
            write(i).start()
            if i + _NBUF - 1 < nchunks:
                if i >= 1:
                    write(i - 1).wait()
                read(i + _NBUF - 1).start()
        for j in range(max(0, nchunks - _NBUF), nchunks):
            write(j).wait()

    return _copy(src)


# speedup vs baseline: 1.0136x; 1.0136x over previous
import functools

import jax
import jax.numpy as jnp
from jax import lax
from jax.experimental import pallas as pl
from jax.experimental.pallas import tpu as pltpu
from jax.experimental.pallas import tpu_sc as plsc

_INFO = plsc.get_sparse_core_info()
_NC, _NS = _INFO.num_cores, _INFO.num_subcores
_NW = _NC * _NS
_CAP = 40
_NBUF = 3


def _chunks(rows):
    out, off = [], 0
    while off < rows:
        sz = min(_CAP, rows - off)
        out.append((off, sz))
        off += sz
    return out


def kernel(x, pos_emb):
    S = x.shape[0]
    D = pos_emb.shape[1]
    src = pos_emb[:S]
    rows_per_w = S // _NW
    chunks = _chunks(rows_per_w)
    nchunks = len(chunks)
    mesh = plsc.VectorSubcoreMesh(core_axis_name="c", subcore_axis_name="s")

    @functools.partial(
        pl.kernel,
        out_type=jax.ShapeDtypeStruct((S, 1, D), jnp.float32),
        mesh=mesh,
        scratch_types=[
            pltpu.VMEM((_NBUF, _CAP, D), jnp.float32),
            pltpu.SemaphoreType.DMA((_NBUF,)),
            pltpu.SemaphoreType.DMA((_NBUF,)),
        ],
    )
    def _copy(src_hbm, out_hbm, buf, rsem, wsem):
        wid = lax.axis_index("s") * _NC + lax.axis_index("c")
        base = wid * rows_per_w

        def read(i):
            off, sz = chunks[i]
            return pltpu.make_async_copy(
                src_hbm.at[pl.ds(base + off, sz)],
                buf.at[i % _NBUF, pl.ds(0, sz)],
                rsem.at[i % _NBUF],
            )

        def write(i):
            off, sz = chunks[i]
            return pltpu.make_async_copy(
                buf.at[i % _NBUF, pl.ds(0, sz)],
                out_hbm.at[pl.ds(base + off, sz), 0],
                wsem.at[i % _NBUF],
            )

        for i in range(min(_NBUF - 1, nchunks)):
            read(i).start()
        for i in range(nchunks):
            read(i).wait()
            write(i).start()
            if i + _NBUF - 1 < nchunks:
                if i >= 1:
                    write(i - 1).wait()
                read(i + _NBUF - 1).start()
        for j in range(max(0, nchunks - _NBUF), nchunks):
            write(j).wait()

    return _copy(src)
